# baseline (speedup 1.0000x reference)
import functools

import jax
import jax.numpy as jnp
from jax.experimental import pallas as pl
from jax.experimental.pallas import tpu as pltpu
from jax.experimental.pallas import tpu_sc as plsc

E = 64
K = 8
CAPACITY_FACTOR = 1.25


def _router_kernel(x_ref, w_ref, idx_ref, wts_ref, keep_ref, gidx_ref,
                   loss_ref, bits_scr, idx_scr, imp_scr, *, capacity, blk,
                   grid):
    i = pl.program_id(0)
    x = x_ref[...]
    w = w_ref[...]
    logits = jax.lax.dot_general(
        w, x, (((1,), (1,)), ((), ())), preferred_element_type=jnp.float32
    )
    m = jnp.max(logits, axis=0, keepdims=True)
    p = jnp.exp(logits - m)
    p = p / jnp.sum(p, axis=0, keepdims=True)

    @pl.when(i == 0)
    def _():
        imp_scr[...] = jnp.zeros_like(imp_scr)

    imp_scr[...] += p

    row = jax.lax.broadcasted_iota(jnp.int32, p.shape, 0)
    probs = p
    idxs = []
    vals = []
    for _ in range(K):
        mk = jnp.max(probs, axis=0, keepdims=True)
        ik = jnp.min(jnp.where(probs == mk, row, E), axis=0, keepdims=True)
        idxs.append(ik)
        vals.append(mk)
        probs = jnp.where(row == ik, -1.0, probs)

    idx = jnp.concatenate(idxs, axis=0)
    wts = jnp.concatenate(vals, axis=0)
    wsum = jnp.sum(wts, axis=0, keepdims=True)
    wn = wts / wsum
    idx_ref[...] = idx
    wts_ref[...] = wn
    idx_scr[:, pl.ds(i * blk, blk)] = idx

    dense = jnp.zeros_like(p)
    for k in range(K):
        dense = dense + jnp.where(row == idxs[k], wn[k:k + 1, :], 0.0)
    bits_scr[:, pl.ds(i * blk, blk)] = jax.lax.bitcast_convert_type(
        dense, jnp.int32)

    @pl.when(i == grid - 1)
    def _epilogue():
        bits = bits_scr[...]

        lo = jnp.zeros((E, 1), jnp.int32)
        hi = jnp.full((E, 1), jnp.int32(0x40000000))

        def p1(_, state):
            lo, hi = state
            mid = lo + (hi - lo) // 2
            cnt = jnp.sum((bits >= mid).astype(jnp.int32), axis=1,
                          keepdims=True)
            pred = cnt >= capacity
            return jnp.where(pred, mid, lo), jnp.where(pred, hi, mid)

        lo, hi = jax.lax.fori_loop(0, 30, p1, (lo, hi))
        t = lo

        gt = bits > t
        eq = bits == t
        count_gt = jnp.sum(gt.astype(jnp.int32), axis=1, keepdims=True)
        count_eq = jnp.sum(eq.astype(jnp.int32), axis=1, keepdims=True)
        remaining = capacity - count_gt

        cols = jax.lax.broadcasted_iota(jnp.int32, bits.shape, 1)
        ntok = bits.shape[1]

        def do_p2(_):
            eq_cols = jnp.where(eq, cols, ntok)

            def p2(_, state):
                lo2, hi2 = state
                mid = lo2 + (hi2 - lo2 + 1) // 2
                cnt = jnp.sum((eq_cols <= mid).astype(jnp.int32), axis=1,
                              keepdims=True)
                pred = cnt >= remaining
                return jnp.where(pred, lo2, mid), jnp.where(pred, mid, hi2)

            return jax.lax.fori_loop(
                0, 13, p2,
                (jnp.full((E, 1), -1, jnp.int32),
                 jnp.full((E, 1), ntok - 1, jnp.int32)))[1]

        need_p2 = jnp.any(count_eq != remaining)
        hi2 = jax.lax.cond(
            need_p2, do_p2,
            lambda _: jnp.full((E, 1), ntok - 1, jnp.int32), None)
        keep_ref[...] = (gt | (eq & (cols <= hi2))).astype(jnp.float32)

        idxf = idx_scr[...]
        tok = jax.lax.broadcasted_iota(jnp.int32, idxf.shape, 1)
        gidx_ref[...] = idxf * ntok + tok

        imp = jnp.sum(imp_scr[...], axis=1, keepdims=True)
        impn = imp / jnp.sum(imp)
        load = jnp.sum((bits > 0).astype(jnp.float32), axis=1, keepdims=True)
        loadn = load / jnp.sum(load)
        loss_ref[...] = E * jnp.sum(impn * loadn, axis=0, keepdims=True)


def _sc_gather(total):
    info = plsc.get_sparse_core_info()
    nw = info.num_cores * info.num_subcores
    b_per_w = total // nw
    mesh = plsc.VectorSubcoreMesh(core_axis_name="c", subcore_axis_name="s")

    @functools.partial(
        pl.kernel, mesh=mesh,
        out_type=jax.ShapeDtypeStruct((total,), jnp.float32),
        scratch_types=[
            pltpu.VMEM((b_per_w,), jnp.int32),
            pltpu.VMEM((b_per_w,), jnp.float32),
            pltpu.SemaphoreType.DMA,
        ],
    )
    def gather(keep_hbm, gidx_hbm, out_hbm, idx_v, vals_v, sem):
        wid = jax.lax.axis_index("s") * info.num_cores + jax.lax.axis_index("c")
        base = wid * b_per_w
        pltpu.sync_copy(gidx_hbm.at[pl.ds(base, b_per_w)], idx_v)
        pltpu.async_copy(keep_hbm.at[idx_v], vals_v, sem).wait()
        pltpu.sync_copy(vals_v, out_hbm.at[pl.ds(base, b_per_w)])

    return gather


def kernel(x, gate_w):
    batch, seq, dim = x.shape
    tokens = batch * seq
    capacity = int(tokens * K / E * CAPACITY_FACTOR)
    xt = x.reshape(tokens, dim)

    blk = 1024
    grid = tokens // blk
    idx, wts, keep, gidx, loss = pl.pallas_call(
        functools.partial(_router_kernel, capacity=capacity, blk=blk,
                          grid=grid),
        grid=(grid,),
        in_specs=[
            pl.BlockSpec((blk, dim), lambda i: (i, 0)),
            pl.BlockSpec((E, dim), lambda i: (0, 0)),
        ],
        out_specs=[
            pl.BlockSpec((K, blk), lambda i: (0, i)),
            pl.BlockSpec((K, blk), lambda i: (0, i)),
            pl.BlockSpec((E, tokens), lambda i: (0, 0)),
            pl.BlockSpec((K, tokens), lambda i: (0, 0)),
            pl.BlockSpec((1, 1), lambda i: (0, 0)),
        ],
        out_shape=[
            jax.ShapeDtypeStruct((K, tokens), jnp.int32),
            jax.ShapeDtypeStruct((K, tokens), jnp.float32),
            jax.ShapeDtypeStruct((E, tokens), jnp.float32),
            jax.ShapeDtypeStruct((K, tokens), jnp.int32),
            jax.ShapeDtypeStruct((1, 1), jnp.float32),
        ],
        scratch_shapes=[
            pltpu.VMEM((E, tokens), jnp.int32),
            pltpu.VMEM((K, tokens), jnp.int32),
            pltpu.VMEM((E, blk), jnp.float32),
        ],
    )(xt, gate_w)

    mask = _sc_gather(K * tokens)(keep.reshape(-1), gidx.reshape(-1))
    mask = mask.reshape(K, tokens)

    return (
        idx.T.reshape(batch, seq, K),
        wts.T.reshape(batch, seq, K),
        loss[0, 0],
        mask.T.reshape(batch, seq, K),
    )


# speedup vs baseline: 1.3353x; 1.3353x over previous
import functools

import jax
import jax.numpy as jnp
from jax.experimental import pallas as pl
from jax.experimental.pallas import tpu as pltpu

E = 64
K = 8
CAPACITY_FACTOR = 1.25


def _router_kernel(x_ref, w_ref, idx_ref, wts_ref, mask_ref, loss_ref,
                   bits_scr, bits16_scr, idx_scr, imp_scr, *, capacity, blk,
                   grid):
    i = pl.program_id(0)
    x = x_ref[...]
    w = w_ref[...]
    logits = jax.lax.dot_general(
        w, x, (((1,), (1,)), ((), ())), preferred_element_type=jnp.float32
    )
    m = jnp.max(logits, axis=0, keepdims=True)
    p = jnp.exp(logits - m)
    p = p / jnp.sum(p, axis=0, keepdims=True)

    @pl.when(i == 0)
    def _():
        imp_scr[...] = jnp.zeros_like(imp_scr)

    imp_scr[...] += p

    row = jax.lax.broadcasted_iota(jnp.int32, p.shape, 0)
    probs = p
    idxs = []
    vals = []
    for _ in range(K):
        mk = jnp.max(probs, axis=0, keepdims=True)
        ik = jnp.min(jnp.where(probs == mk, row, E), axis=0, keepdims=True)
        idxs.append(ik)
        vals.append(mk)
        probs = jnp.where(row == ik, -1.0, probs)

    idx = jnp.concatenate(idxs, axis=0)
    wts = jnp.concatenate(vals, axis=0)
    wsum = jnp.sum(wts, axis=0, keepdims=True)
    wn = wts / wsum
    idx_ref[...] = idx
    wts_ref[...] = wn
    idx_scr[:, pl.ds(i * blk, blk)] = idx

    dense = jnp.zeros_like(p)
    for k in range(K):
        dense = dense + jnp.where(row == idxs[k], wn[k:k + 1, :], 0.0)
    dbits = jax.lax.bitcast_convert_type(dense, jnp.int32)
    bits_scr[:, pl.ds(i * blk, blk)] = dbits
    bits16_scr[:, pl.ds(i * blk, blk)] = (dbits >> 16).astype(jnp.int16)

    @pl.when(i == grid - 1)
    def _epilogue():
        bits = bits_scr[...]

        bits16 = bits16_scr[...]
        lo16 = jnp.zeros((E, 1), jnp.int32)
        hi16 = jnp.full((E, 1), jnp.int32(0x4000))

        def p1a(_, state):
            lo, hi = state
            mid = lo + (hi - lo) // 2
            c = (bits16 >= mid.astype(jnp.int16)).astype(jnp.int16)
            wdt = c.shape[1]
            while wdt > 128:
                half = wdt // 2
                c = c[:, :half] + c[:, half:wdt]
                wdt = half
            cnt = jnp.sum(c.astype(jnp.int32), axis=1, keepdims=True)
            pred = cnt >= capacity
            return jnp.where(pred, mid, lo), jnp.where(pred, hi, mid)

        lo16, hi16 = jax.lax.fori_loop(0, 15, p1a, (lo16, hi16))
        lo = lo16 << 16
        hi = lo + (1 << 16)

        def p1(_, state):
            lo, hi = state
            mid = lo + (hi - lo) // 2
            cnt = jnp.sum((bits >= mid).astype(jnp.int32), axis=1,
                          keepdims=True)
            pred = cnt >= capacity
            return jnp.where(pred, mid, lo), jnp.where(pred, hi, mid)

        lo, hi = jax.lax.fori_loop(0, 16, p1, (lo, hi))
        t = lo

        gt = bits > t
        eq = bits == t
        count_gt = jnp.sum(gt.astype(jnp.int32), axis=1, keepdims=True)
        count_eq = jnp.sum(eq.astype(jnp.int32), axis=1, keepdims=True)
        remaining = capacity - count_gt

        cols = jax.lax.broadcasted_iota(jnp.int32, bits.shape, 1)
        ntok = bits.shape[1]

        def do_p2(_):
            eq_cols = jnp.where(eq, cols, ntok)

            def p2(_, state):
                lo2, hi2 = state
                mid = lo2 + (hi2 - lo2 + 1) // 2
                cnt = jnp.sum((eq_cols <= mid).astype(jnp.int32), axis=1,
                              keepdims=True)
                pred = cnt >= remaining
                return jnp.where(pred, lo2, mid), jnp.where(pred, mid, hi2)

            return jax.lax.fori_loop(
                0, 13, p2,
                (jnp.full((E, 1), -1, jnp.int32),
                 jnp.full((E, 1), ntok - 1, jnp.int32)))[1]

        need_p2 = jnp.any(count_eq != remaining)
        hi2 = jax.lax.cond(
            need_p2, do_p2,
            lambda _: jnp.full((E, 1), ntok - 1, jnp.int32), None)
        keep = (gt | (eq & (cols <= hi2))).astype(jnp.bfloat16)

        idxf = idx_scr[...]
        rows = []
        for k in range(K):
            e = idxf[k:k + 1, :]
            v = keep
            h = E // 2
            while h >= 1:
                v = jnp.where((e & h) != 0, v[h:2 * h, :], v[:h, :])
                h //= 2
            rows.append(v)
        mask_ref[...] = jnp.concatenate(rows, axis=0).astype(jnp.float32)

        imp = jnp.sum(imp_scr[...], axis=1, keepdims=True)
        impn = imp / jnp.sum(imp)
        load = jnp.sum((bits > 0).astype(jnp.float32), axis=1, keepdims=True)
        loadn = load / jnp.sum(load)
        loss_ref[...] = E * jnp.sum(impn * loadn, axis=0, keepdims=True)


def kernel(x, gate_w):
    batch, seq, dim = x.shape
    tokens = batch * seq
    capacity = int(tokens * K / E * CAPACITY_FACTOR)
    xt = x.reshape(tokens, dim)

    blk = 1024
    grid = tokens // blk
    idx, wts, mask, loss = pl.pallas_call(
        functools.partial(_router_kernel, capacity=capacity, blk=blk,
                          grid=grid),
        grid=(grid,),
        in_specs=[
            pl.BlockSpec((blk, dim), lambda i: (i, 0)),
            pl.BlockSpec((E, dim), lambda i: (0, 0)),
        ],
        out_specs=[
            pl.BlockSpec((K, blk), lambda i: (0, i)),
            pl.BlockSpec((K, blk), lambda i: (0, i)),
            pl.BlockSpec((K, tokens), lambda i: (0, 0)),
            pl.BlockSpec((1, 1), lambda i: (0, 0)),
        ],
        out_shape=[
            jax.ShapeDtypeStruct((K, tokens), jnp.int32),
            jax.ShapeDtypeStruct((K, tokens), jnp.float32),
            jax.ShapeDtypeStruct((K, tokens), jnp.float32),
            jax.ShapeDtypeStruct((1, 1), jnp.float32),
        ],
        scratch_shapes=[
            pltpu.VMEM((E, tokens), jnp.int32),
            pltpu.VMEM((E, tokens), jnp.int16),
            pltpu.VMEM((K, tokens), jnp.int32),
            pltpu.VMEM((E, blk), jnp.float32),
        ],
    )(xt, gate_w)

    return (
        idx.T.reshape(batch, seq, K),
        wts.T.reshape(batch, seq, K),
        loss[0, 0],
        mask.T.reshape(batch, seq, K),
    )


# speedup vs baseline: 1.4491x; 1.0852x over previous
import functools

import jax
import jax.numpy as jnp
from jax.experimental import pallas as pl
from jax.experimental.pallas import tpu as pltpu

E = 64
K = 8
CAPACITY_FACTOR = 1.25


def _router_kernel(x_ref, w_ref, idx_ref, wts_ref, mask_ref, loss_ref,
                   bits_scr, idx_scr, imp_scr, *, capacity, blk, grid):
    i = pl.program_id(0)
    x = x_ref[...]
    w = w_ref[...]
    logits = jax.lax.dot_general(
        w, x, (((1,), (1,)), ((), ())), preferred_element_type=jnp.float32
    )
    m = jnp.max(logits, axis=0, keepdims=True)
    p = jnp.exp(logits - m)
    p = p / jnp.sum(p, axis=0, keepdims=True)

    @pl.when(i == 0)
    def _():
        imp_scr[...] = jnp.zeros_like(imp_scr)

    imp_scr[...] += p

    row = jax.lax.broadcasted_iota(jnp.int32, p.shape, 0)
    probs = p
    idxs = []
    vals = []
    for _ in range(K):
        mk = jnp.max(probs, axis=0, keepdims=True)
        ik = jnp.min(jnp.where(probs == mk, row, E), axis=0, keepdims=True)
        idxs.append(ik)
        vals.append(mk)
        probs = jnp.where(row == ik, -1.0, probs)

    idx = jnp.concatenate(idxs, axis=0)
    wts = jnp.concatenate(vals, axis=0)
    wsum = jnp.sum(wts, axis=0, keepdims=True)
    wn = wts / wsum
    idx_ref[...] = idx
    wts_ref[...] = wn
    idx_scr[:, pl.ds(i * blk, blk)] = idx

    dense = jnp.zeros_like(p)
    for k in range(K):
        dense = dense + jnp.where(row == idxs[k], wn[k:k + 1, :], 0.0)
    bits_scr[:, pl.ds(i * blk, blk)] = jax.lax.bitcast_convert_type(
        dense, jnp.int32)

    @pl.when(i == grid - 1)
    def _epilogue():
        bits = bits_scr[...]

        lo = jnp.zeros((E, 1), jnp.int32)
        hi = jnp.full((E, 1), jnp.int32(0x40000000))

        def p1(_, state):
            lo, hi = state
            mid = lo + (hi - lo) // 2
            cnt = jnp.sum((bits >= mid).astype(jnp.int32), axis=1,
                          keepdims=True)
            pred = cnt >= capacity
            return jnp.where(pred, mid, lo), jnp.where(pred, hi, mid)

        lo, hi = jax.lax.fori_loop(0, 30, p1, (lo, hi))
        t = lo

        gt = bits > t
        eq = bits == t
        count_gt = jnp.sum(gt.astype(jnp.int32), axis=1, keepdims=True)
        count_eq = jnp.sum(eq.astype(jnp.int32), axis=1, keepdims=True)
        remaining = capacity - count_gt

        cols = jax.lax.broadcasted_iota(jnp.int32, bits.shape, 1)
        ntok = bits.shape[1]

        def do_p2(_):
            eq_cols = jnp.where(eq, cols, ntok)

            def p2(_, state):
                lo2, hi2 = state
                mid = lo2 + (hi2 - lo2 + 1) // 2
                cnt = jnp.sum((eq_cols <= mid).astype(jnp.int32), axis=1,
                              keepdims=True)
                pred = cnt >= remaining
                return jnp.where(pred, lo2, mid), jnp.where(pred, mid, hi2)

            return jax.lax.fori_loop(
                0, 13, p2,
                (jnp.full((E, 1), -1, jnp.int32),
                 jnp.full((E, 1), ntok - 1, jnp.int32)))[1]

        need_p2 = jnp.any((count_eq > remaining) & (t > 0))
        hi2 = jax.lax.cond(
            need_p2, do_p2,
            lambda _: jnp.full((E, 1), ntok - 1, jnp.int32), None)
        keep = (gt | (eq & (cols <= hi2))).astype(jnp.bfloat16)

        idxf = idx_scr[...]
        rows = []
        for k in range(K):
            e = idxf[k:k + 1, :]
            v = keep
            h = E // 2
            while h >= 1:
                v = jnp.where((e & h) != 0, v[h:2 * h, :], v[:h, :])
                h //= 2
            rows.append(v)
        mask_ref[...] = jnp.concatenate(rows, axis=0).astype(jnp.float32)

        imp = jnp.sum(imp_scr[...], axis=1, keepdims=True)
        impn = imp / jnp.sum(imp)
        load = jnp.sum((bits > 0).astype(jnp.float32), axis=1, keepdims=True)
        loadn = load / jnp.sum(load)
        loss_ref[...] = E * jnp.sum(impn * loadn, axis=0, keepdims=True)


def kernel(x, gate_w):
    batch, seq, dim = x.shape
    tokens = batch * seq
    capacity = int(tokens * K / E * CAPACITY_FACTOR)
    xt = x.reshape(tokens, dim)

    blk = 1024
    grid = tokens // blk
    idx, wts, mask, loss = pl.pallas_call(
        functools.partial(_router_kernel, capacity=capacity, blk=blk,
                          grid=grid),
        grid=(grid,),
        in_specs=[
            pl.BlockSpec((blk, dim), lambda i: (i, 0)),
            pl.BlockSpec((E, dim), lambda i: (0, 0)),
        ],
        out_specs=[
            pl.BlockSpec((K, blk), lambda i: (0, i)),
            pl.BlockSpec((K, blk), lambda i: (0, i)),
            pl.BlockSpec((K, tokens), lambda i: (0, 0)),
            pl.BlockSpec((1, 1), lambda i: (0, 0)),
        ],
        out_shape=[
            jax.ShapeDtypeStruct((K, tokens), jnp.int32),
            jax.ShapeDtypeStruct((K, tokens), jnp.float32),
            jax.ShapeDtypeStruct((K, tokens), jnp.float32),
            jax.ShapeDtypeStruct((1, 1), jnp.float32),
        ],
        scratch_shapes=[
            pltpu.VMEM((E, tokens), jnp.int32),
            pltpu.VMEM((K, tokens), jnp.int32),
            pltpu.VMEM((E, blk), jnp.float32),
        ],
    )(xt, gate_w)

    return (
        idx.T.reshape(batch, seq, K),
        wts.T.reshape(batch, seq, K),
        loss[0, 0],
        mask.T.reshape(batch, seq, K),
    )
